# baseline (speedup 1.0000x reference)
import functools

import jax
import jax.numpy as jnp
from jax import lax
from jax.experimental import pallas as pl
from jax.experimental.pallas import tpu as pltpu
from jax.experimental.pallas import tpu_sc as plsc


DIM = 64
N_CODES = 1024
PIX = 1024
BPR = 4


def _argmin_body(x_ref, embed_ref, ind_ref):
    embed = embed_ref[...]
    embt = jnp.transpose(embed)
    esq = jnp.sum(embt * embt, axis=1, keepdims=True)

    for j in range(BPR):
        x = x_ref[j]
        scores2 = jax.lax.dot_general(
            embed, -2.0 * x, (((0,), (0,)), ((), ())),
            preferred_element_type=jnp.float32)
        xsq = jnp.sum(x * x, axis=0, keepdims=True)
        dist = (xsq + scores2) + esq
        ind_ref[j] = jnp.argmin(dist, axis=0, keepdims=True)


def _gather_sc(table_hbm, idx_hbm, out_hbm, idx_v, rows_v, sem):
    wid = lax.axis_index("s") * 2 + lax.axis_index("c")
    for j in range(2):
        base = wid * 512 + j * 256
        pltpu.sync_copy(idx_hbm.at[pl.ds(base, 256)], idx_v)
        pltpu.async_copy(table_hbm.at[idx_v], rows_v, sem).wait()
        pltpu.sync_copy(rows_v, out_hbm.at[pl.ds(base, 256)])


def _finish_body(x_ref, rows_ref, quant_ref, diff_ref):
    x = x_ref[0]
    quant = jnp.transpose(rows_ref[0])
    quant_ref[0] = quant
    diff_ref[0] = quant - x


def kernel(inputs, embed):
    b, c, h, w = inputs.shape
    x = inputs.reshape(b, c, h * w)

    ind = pl.pallas_call(
        _argmin_body,
        grid=(b // BPR,),
        in_specs=[
            pl.BlockSpec((BPR, c, h * w), lambda i: (i, 0, 0)),
            pl.BlockSpec((DIM, N_CODES), lambda i: (0, 0)),
        ],
        out_specs=pl.BlockSpec((BPR, 1, h * w), lambda i: (i, 0, 0)),
        out_shape=jax.ShapeDtypeStruct((b, 1, h * w), jnp.int32),
    )(x, embed)

    embt = embed.T
    idx_flat = ind.reshape(b * h * w)

    mesh = plsc.VectorSubcoreMesh(core_axis_name="c", subcore_axis_name="s")
    gather = functools.partial(
        pl.kernel, _gather_sc, mesh=mesh,
        compiler_params=pltpu.CompilerParams(use_tc_tiling_on_sc=False),
        out_type=jax.ShapeDtypeStruct((b * h * w, DIM), jnp.float32),
        scratch_types=[
            pltpu.VMEM((256,), jnp.int32),
            pltpu.VMEM((256, DIM), jnp.float32),
            pltpu.SemaphoreType.DMA,
        ],
    )()
    qrows = gather(embt, idx_flat)

    quant, diff = pl.pallas_call(
        _finish_body,
        grid=(b,),
        in_specs=[
            pl.BlockSpec((1, c, h * w), lambda i: (i, 0, 0)),
            pl.BlockSpec((1, h * w, DIM), lambda i: (i, 0, 0)),
        ],
        out_specs=[
            pl.BlockSpec((1, c, h * w), lambda i: (i, 0, 0)),
            pl.BlockSpec((1, c, h * w), lambda i: (i, 0, 0)),
        ],
        out_shape=[
            jax.ShapeDtypeStruct((b, c, h * w), jnp.float32),
            jax.ShapeDtypeStruct((b, c, h * w), jnp.float32),
        ],
    )(x, qrows.reshape(b, h * w, DIM))

    return (quant.reshape(b, c, h, w),
            diff.reshape(b, c, h, w),
            ind.reshape(b, h, w))


# speedup vs baseline: 2.2206x; 2.2206x over previous
import jax
import jax.numpy as jnp
from jax.experimental import pallas as pl


DIM = 64
N_CODES = 1024
PIX = 1024
BPR = 4


def _vq_body(x_ref, embed_ref, quant_ref, diff_ref, ind_ref):
    embed = embed_ref[...]
    embt = jnp.transpose(embed)
    esq = jnp.sum(embt * embt, axis=1, keepdims=True)
    embbf = embed.astype(jnp.bfloat16)
    code_iota = jax.lax.broadcasted_iota(jnp.int32, (N_CODES, PIX), 0)

    for j in range(BPR):
        x = x_ref[j]

        scores2 = jax.lax.dot_general(
            embed, -2.0 * x, (((0,), (0,)), ((), ())),
            preferred_element_type=jnp.float32)
        xsq = jnp.sum(x * x, axis=0, keepdims=True)
        dist = (xsq + scores2) + esq

        ind = jnp.argmin(dist, axis=0, keepdims=True)
        ind_ref[j] = ind

        onehot = (code_iota == ind).astype(jnp.bfloat16)
        quant = jax.lax.dot_general(
            embbf, onehot, (((1,), (0,)), ((), ())),
            preferred_element_type=jnp.float32)
        quant_ref[j] = quant
        diff_ref[j] = quant - x


def kernel(inputs, embed):
    b, c, h, w = inputs.shape
    x = inputs.reshape(b, c, h * w)

    quant, diff, ind = pl.pallas_call(
        _vq_body,
        grid=(b // BPR,),
        in_specs=[
            pl.BlockSpec((BPR, c, h * w), lambda i: (i, 0, 0)),
            pl.BlockSpec((DIM, N_CODES), lambda i: (0, 0)),
        ],
        out_specs=[
            pl.BlockSpec((BPR, c, h * w), lambda i: (i, 0, 0)),
            pl.BlockSpec((BPR, c, h * w), lambda i: (i, 0, 0)),
            pl.BlockSpec((BPR, 1, h * w), lambda i: (i, 0, 0)),
        ],
        out_shape=[
            jax.ShapeDtypeStruct((b, c, h * w), jnp.float32),
            jax.ShapeDtypeStruct((b, c, h * w), jnp.float32),
            jax.ShapeDtypeStruct((b, 1, h * w), jnp.int32),
        ],
    )(x, embed)

    return (quant.reshape(b, c, h, w),
            diff.reshape(b, c, h, w),
            ind.reshape(b, h, w))
